# baseline (speedup 1.0000x reference)
import jax
import jax.numpy as jnp
import numpy as np
from jax.experimental import pallas as pl

B = 1024
D = 512
H = 8
DH = D // H


def _mm_t(a, w):
    return jax.lax.dot_general(a, w, (((1,), (1,)), ((), ())),
                               preferred_element_type=jnp.float32)


def _fused_kernel(x_ref, W1_ref, b1_ref, W2_ref, b2_ref,
                  Wq_ref, bq_ref, Wk_ref, bk_ref, Wv_ref, bv_ref,
                  Wo_ref, bo_ref, W3p_ref, W3a_ref, b3_ref,
                  W4_ref, b4_ref, out_ref):
    bf16 = jnp.bfloat16
    x = x_ref[...]
    h = jnp.maximum(_mm_t(x, W1_ref[...]) + b1_ref[...], 0.0).astype(bf16)
    patterns = jnp.maximum(_mm_t(h, W2_ref[...]) + b2_ref[...], 0.0).astype(bf16)

    q = _mm_t(x, Wq_ref[...]) + bq_ref[...]
    k = (_mm_t(x, Wk_ref[...]) + bk_ref[...]).astype(bf16)
    v = (_mm_t(x, Wv_ref[...]) + bv_ref[...]).astype(bf16)
    scale = np.float32(1.0 / np.sqrt(DH))
    q = (q * scale).astype(bf16)

    head_outs = []
    for hh in range(H):
        qh = q[:, hh * DH:(hh + 1) * DH]
        kh = k[:, hh * DH:(hh + 1) * DH]
        vh = v[:, hh * DH:(hh + 1) * DH]
        s = jax.lax.dot_general(qh, kh, (((1,), (1,)), ((), ())),
                                preferred_element_type=jnp.float32)
        m = jnp.max(s, axis=-1, keepdims=True)
        e = jnp.exp(s - m)
        r = 1.0 / jnp.sum(e, axis=-1, keepdims=True)
        o = jnp.dot(e.astype(bf16), vh, preferred_element_type=jnp.float32)
        head_outs.append(o * r)
    att = jnp.concatenate(head_outs, axis=-1).astype(bf16)
    attended = (_mm_t(att, Wo_ref[...]) + bo_ref[...]).astype(bf16)

    h2 = jnp.maximum(_mm_t(patterns, W3p_ref[...])
                     + _mm_t(attended, W3a_ref[...]) + b3_ref[...], 0.0)
    out_ref[...] = jnp.tanh(_mm_t(h2.astype(bf16), W4_ref[...]) + b4_ref[...])


def kernel(sensory_input, W1, b1, W2, b2, Wq, bq, Wk, bk, Wv, bv, Wo, bo, W3, b3, W4, b4):
    bf16 = jnp.bfloat16
    W3p = W3[:, :128]
    W3a = W3[:, 128:]
    args = (sensory_input.astype(bf16),
            W1.astype(bf16), b1.reshape(1, -1), W2.astype(bf16), b2.reshape(1, -1),
            Wq.astype(bf16), bq.reshape(1, -1), Wk.astype(bf16), bk.reshape(1, -1),
            Wv.astype(bf16), bv.reshape(1, -1), Wo.astype(bf16), bo.reshape(1, -1),
            W3p.astype(bf16), W3a.astype(bf16), b3.reshape(1, -1),
            W4.astype(bf16), b4.reshape(1, -1))
    return pl.pallas_call(
        _fused_kernel,
        out_shape=jax.ShapeDtypeStruct((B, D), jnp.float32),
    )(*args)


# speedup vs baseline: 1.0597x; 1.0597x over previous
import jax
import jax.numpy as jnp
import numpy as np
from jax.experimental import pallas as pl

B = 1024
D = 512
H = 8
DH = D // H


def _mm_t(a, w):
    return jax.lax.dot_general(a, w, (((1,), (1,)), ((), ())),
                               preferred_element_type=jnp.float32)


def _fused_kernel(x_ref, W1_ref, b1_ref, W2_ref, b2_ref,
                  Wq_ref, bq_ref, Wk_ref, bk_ref, Wv_ref, bv_ref,
                  Wo_ref, bo_ref, W3p_ref, W3a_ref, b3_ref,
                  W4_ref, b4_ref, out_ref):
    x = x_ref[...]
    h = jnp.maximum(_mm_t(x, W1_ref[...]) + b1_ref[...], 0.0).astype(jnp.bfloat16)
    patterns = jnp.maximum(_mm_t(h, W2_ref[...]) + b2_ref[...], 0.0).astype(jnp.bfloat16)

    bf16 = jnp.bfloat16
    scale = np.float32(1.0 / np.sqrt(DH))
    q = ((_mm_t(x, Wq_ref[...]) + bq_ref[...]) * scale).astype(bf16)
    k = (_mm_t(x, Wk_ref[...]) + bk_ref[...]).astype(bf16)
    v = (_mm_t(x, Wv_ref[...]) + bv_ref[...]).astype(bf16)

    col = jax.lax.broadcasted_iota(jnp.int32, (B, DH), 1)
    ones_blk = (col == 0).astype(bf16)

    head_outs = []
    for hh in range(H):
        qh = q[:, hh * DH:(hh + 1) * DH]
        kh = k[:, hh * DH:(hh + 1) * DH]
        vh = jnp.concatenate([v[:, hh * DH:(hh + 1) * DH], ones_blk], axis=-1)
        s = jax.lax.dot_general(qh, kh, (((1,), (1,)), ((), ())),
                                preferred_element_type=jnp.float32).astype(bf16)
        m = jnp.max(s, axis=-1, keepdims=True)
        e = jnp.exp(s - m)
        o2 = jnp.dot(e, vh, preferred_element_type=jnp.float32)
        r = 1.0 / o2[:, DH:DH + 1]
        head_outs.append(o2[:, :DH] * r)
    att = jnp.concatenate(head_outs, axis=-1).astype(jnp.bfloat16)
    attended = (_mm_t(att, Wo_ref[...]) + bo_ref[...]).astype(jnp.bfloat16)

    h2 = jnp.maximum(_mm_t(patterns, W3p_ref[...])
                     + _mm_t(attended, W3a_ref[...]) + b3_ref[...], 0.0)
    out_ref[...] = jnp.tanh(_mm_t(h2.astype(jnp.bfloat16), W4_ref[...]) + b4_ref[...])


def kernel(sensory_input, W1, b1, W2, b2, Wq, bq, Wk, bk, Wv, bv, Wo, bo, W3, b3, W4, b4):
    W3p = W3[:, :128]
    W3a = W3[:, 128:]
    bf16 = jnp.bfloat16
    args = (sensory_input.astype(bf16),
            W1.astype(bf16), b1.reshape(1, -1), W2.astype(bf16), b2.reshape(1, -1),
            Wq.astype(bf16), bq.reshape(1, -1), Wk.astype(bf16), bk.reshape(1, -1),
            Wv.astype(bf16), bv.reshape(1, -1), Wo.astype(bf16), bo.reshape(1, -1),
            W3p.astype(bf16), W3a.astype(bf16), b3.reshape(1, -1),
            W4.astype(bf16), b4.reshape(1, -1))
    return pl.pallas_call(
        _fused_kernel,
        out_shape=jax.ShapeDtypeStruct((B, D), jnp.float32),
    )(*args)
